# baseline (speedup 1.0000x reference)
import jax
import jax.numpy as jnp
from jax.experimental import pallas as pl

N = 50000
E = 800000
H = 8
C = 8
OUT = H * C


def _relu_kernel(x_ref, o_ref):
    o_ref[...] = jnp.maximum(x_ref[...], 0.0)


def _pallas_relu(x):
    return pl.pallas_call(
        _relu_kernel,
        out_shape=jax.ShapeDtypeStruct(x.shape, x.dtype),
        grid=(50,),
        in_specs=[pl.BlockSpec((1000, 64), lambda i: (i, 0))],
        out_specs=pl.BlockSpec((1000, 64), lambda i: (i, 0)),
    )(x)


def _add_self_loops_mean(src, dst, ea, n):
    s = jax.ops.segment_sum(ea, dst, num_segments=n)
    c = jax.ops.segment_sum(jnp.ones((ea.shape[0], 1), ea.dtype), dst, num_segments=n)
    m = s / jnp.maximum(c, 1.0)
    loop = jnp.arange(n, dtype=src.dtype)
    return (jnp.concatenate([src, loop]), jnp.concatenate([dst, loop]),
            jnp.concatenate([ea, m], axis=0))


def _gat(x, src, dst, ea, W, a_src, a_dst, We, a_e, b, add_sl, n):
    if add_sl:
        src, dst, ea = _add_self_loops_mean(src, dst, ea, n)
    h = (x @ W).reshape(n, H, C)
    asrc = jnp.sum(h * a_src, axis=-1)
    adst = jnp.sum(h * a_dst, axis=-1)
    e = (ea @ We).reshape(-1, H, C)
    ae_ = jnp.sum(e * a_e, axis=-1)
    alpha = asrc[src] + adst[dst] + ae_
    alpha = jax.nn.leaky_relu(alpha, 0.2)
    amax = jax.ops.segment_max(alpha, dst, num_segments=n)
    amax = jnp.where(jnp.isfinite(amax), amax, 0.0)
    ex = jnp.exp(alpha - amax[dst])
    den = jax.ops.segment_sum(ex, dst, num_segments=n)
    w = ex / (den[dst] + 1e-16)
    msg = h[src] * w[:, :, None]
    out = jax.ops.segment_sum(msg, dst, num_segments=n).reshape(n, H * C)
    return out + b


def kernel(x, edge_index, edge_attr, W1, as1, ad1, We1, ae1, b1,
           W2, as2, ad2, We2, ae2, b2, W3, as3, ad3, We3, ae3, b3):
    src, dst = edge_index[0], edge_index[1]
    h = _gat(x, src, dst, edge_attr, W1, as1, ad1, We1, ae1, b1, False, N)
    h = jax.nn.relu(h)
    h = _gat(h, src, dst, edge_attr, W2, as2, ad2, We2, ae2, b2, True, N)
    h = jax.nn.relu(h)
    h = _gat(h, src, dst, edge_attr, W3, as3, ad3, We3, ae3, b3, True, N)
    return _pallas_relu(h)


# speedup vs baseline: 14.7566x; 14.7566x over previous
import functools

import jax
import jax.numpy as jnp
from jax.experimental import pallas as pl

N = 50000
E = 800000
H = 8
C = 8
OUT = H * C

NC = 2
NP = 50048
RPT = NP // 16
SREC_W = 48
DREC_W = 16
ACC_W = 36
EA_W = 8


def _dense_body(x_ref, w_ref, as_ref, ad_ref, srec_ref, drec_ref, mx_ref):
    c = pl.program_id(0)
    x = x_ref[...]
    nr = x.shape[0]
    h = jnp.dot(x, w_ref[...], preferred_element_type=jnp.float32)
    asrc = jnp.dot(h, as_ref[...], preferred_element_type=jnp.float32)
    adst = jnp.dot(h, ad_ref[...], preferred_element_type=jnp.float32)
    zpad = jnp.zeros((nr, SREC_W - 36), jnp.float32)
    rec0 = jnp.concatenate([h[:, :32], asrc[:, :4], zpad], axis=1)
    rec1 = jnp.concatenate([h[:, 32:], asrc[:, 4:], zpad], axis=1)
    srec_ref[...] = jnp.where(c == 0, rec0, rec1)[None]
    drec_ref[...] = jnp.concatenate(
        [adst, jnp.zeros((nr, DREC_W - H), jnp.float32)], axis=1)
    mx_ref[...] = jnp.concatenate(
        [jnp.max(asrc, axis=0), jnp.max(adst, axis=0)])[None, None]


def _dense(xp, W, As, Ad):
    IN = xp.shape[1]
    return pl.pallas_call(
        _dense_body,
        grid=(NC, NP // RPT),
        in_specs=[
            pl.BlockSpec((RPT, IN), lambda c, i: (i, 0)),
            pl.BlockSpec((IN, OUT), lambda c, i: (0, 0)),
            pl.BlockSpec((OUT, H), lambda c, i: (0, 0)),
            pl.BlockSpec((OUT, H), lambda c, i: (0, 0)),
        ],
        out_specs=[
            pl.BlockSpec((1, RPT, SREC_W), lambda c, i: (c, i, 0)),
            pl.BlockSpec((RPT, DREC_W), lambda c, i: (i, 0)),
            pl.BlockSpec((1, 1, 2 * H), lambda c, i: (i, 0, 0)),
        ],
        out_shape=[
            jax.ShapeDtypeStruct((NC, NP, SREC_W), jnp.float32),
            jax.ShapeDtypeStruct((NP, DREC_W), jnp.float32),
            jax.ShapeDtypeStruct((NP // RPT, 1, 2 * H), jnp.float32),
        ],
    )(xp, W, As, Ad)


def _ea_range_body(ea_ref, mx_ref, mn_ref):
    mx_ref[...] = jnp.max(ea_ref[...], axis=0)[None]
    mn_ref[...] = jnp.min(ea_ref[...], axis=0)[None]


def _ea_range(ea2):
    mx, mn = pl.pallas_call(
        _ea_range_body,
        out_shape=[
            jax.ShapeDtypeStruct((1, 128), jnp.float32),
            jax.ShapeDtypeStruct((1, 128), jnp.float32),
        ],
    )(ea2)
    return jnp.max(mx), jnp.min(mn)


def _edge_ex_body(asrc_ref, adst_ref, ea_ref, b_ref, ex_ref):
    alpha = asrc_ref[...] + adst_ref[...] + ea_ref[...]
    alpha = jnp.maximum(alpha, alpha * 0.2)
    ex_ref[...] = jnp.exp(alpha - b_ref[...])


def _edge_ex(asrc_e, adst_e, ae_e, B):
    EB = 6400
    return pl.pallas_call(
        _edge_ex_body,
        grid=(E // EB,),
        in_specs=[
            pl.BlockSpec((EB, H), lambda i: (i, 0)),
            pl.BlockSpec((EB, H), lambda i: (i, 0)),
            pl.BlockSpec((EB, H), lambda i: (i, 0)),
            pl.BlockSpec((1, H), lambda i: (0, 0)),
        ],
        out_specs=pl.BlockSpec((EB, H), lambda i: (i, 0)),
        out_shape=jax.ShapeDtypeStruct((E, H), jnp.float32),
    )(asrc_e, adst_e, ae_e, B)


def _finalize_body(with_loop, acc_ref, srec_ref, drec_ref, eam_ref, sea_ref,
                   we_ref, b_ref, bias_ref, x_ref, eam_out_ref):
    acc0 = acc_ref[0]
    acc1 = acc_ref[1]
    num = jnp.concatenate([acc0[:, :32], acc1[:, :32]], axis=1)
    den = jnp.concatenate([acc0[:, 32:36], acc1[:, 32:36]], axis=1)
    if with_loop:
        h = jnp.concatenate([srec_ref[0][:, :32], srec_ref[1][:, :32]], axis=1)
        asrc = jnp.concatenate(
            [srec_ref[0][:, 32:36], srec_ref[1][:, 32:36]], axis=1)
        adst = drec_ref[:, :8]
        alpha = asrc + adst + eam_ref[...] * we_ref[...]
        alpha = jnp.maximum(alpha, alpha * 0.2)
        exl = jnp.exp(alpha - b_ref[...])
        nrows = h.shape[0]
        num = num + (h.reshape(nrows, H, C) * exl[:, :, None]).reshape(
            nrows, OUT)
        den = den + exl
    nrows = num.shape[0]
    den64 = jnp.broadcast_to(den[:, :, None], (nrows, H, C)).reshape(nrows, OUT)
    o = num / (den64 + 1e-16) + bias_ref[...]
    x_ref[...] = jnp.maximum(o, 0.0)
    eam_out_ref[...] = sea_ref[:, 0:1] / jnp.maximum(sea_ref[:, 1:2], 1.0)


FRT = 1088


def _finalize(acc, srecs, drec, eam, sea, we2, B2, bias2, with_loop):
    body = functools.partial(_finalize_body, with_loop)
    return pl.pallas_call(
        body,
        grid=(NP // FRT,),
        in_specs=[
            pl.BlockSpec((NC, FRT, ACC_W), lambda i: (0, i, 0)),
            pl.BlockSpec((NC, FRT, SREC_W), lambda i: (0, i, 0)),
            pl.BlockSpec((FRT, DREC_W), lambda i: (i, 0)),
            pl.BlockSpec((FRT, 1), lambda i: (i, 0)),
            pl.BlockSpec((FRT, EA_W), lambda i: (i, 0)),
            pl.BlockSpec((1, H), lambda i: (0, 0)),
            pl.BlockSpec((1, H), lambda i: (0, 0)),
            pl.BlockSpec((1, OUT), lambda i: (0, 0)),
        ],
        out_specs=[
            pl.BlockSpec((FRT, OUT), lambda i: (i, 0)),
            pl.BlockSpec((FRT, 1), lambda i: (i, 0)),
        ],
        out_shape=[
            jax.ShapeDtypeStruct((NP, OUT), jnp.float32),
            jax.ShapeDtypeStruct((NP, 1), jnp.float32),
        ],
    )(acc, srecs, drec, eam, sea, we2, B2, bias2)


def _block_diag_att(a):
    eye = jnp.eye(H, dtype=jnp.float32)
    return (a[:, :, None] * eye[:, None, :]).reshape(OUT, H)


def kernel(x, edge_index, edge_attr, W1, as1, ad1, We1, ae1, b1,
           W2, as2, ad2, We2, ae2, b2, W3, as3, ad3, We3, ae3, b3):
    src = edge_index[0].astype(jnp.int32)
    dst = edge_index[1].astype(jnp.int32)
    ea = edge_attr[:, 0]

    ea_max, ea_min = _ea_range(ea.reshape(E // 128, 128))

    sea_s = jax.ops.segment_sum(ea, dst, num_segments=NP)
    sea_c = jax.ops.segment_sum(jnp.ones((E,), jnp.float32), dst,
                                num_segments=NP)
    sea = jnp.concatenate(
        [sea_s[:, None], sea_c[:, None], jnp.zeros((NP, EA_W - 2))], axis=1)

    xp = jnp.zeros((NP, x.shape[1]), jnp.float32).at[:N].set(x)
    eam = jnp.zeros((NP, 1), jnp.float32)

    layers = [
        (W1, as1, ad1, We1, ae1, b1, False),
        (W2, as2, ad2, We2, ae2, b2, True),
        (W3, as3, ad3, We3, ae3, b3, True),
    ]
    for W, a_s, a_d, We, a_e, bias, with_loop in layers:
        As = _block_diag_att(a_s)
        Ad = _block_diag_att(a_d)
        we = jnp.sum(We.reshape(H, C) * a_e, axis=1)

        srecs, drec, mx3 = _dense(xp, W, As, Ad)
        mx = mx3[:, 0, :]
        bsum = (jnp.max(mx[:, :H], axis=0) + jnp.max(mx[:, H:], axis=0)
                + jnp.maximum(we * ea_max, we * ea_min))
        B = jnp.maximum(bsum, bsum * 0.2)

        h = jnp.concatenate([srecs[0, :, :32], srecs[1, :, :32]], axis=1)
        asrc = jnp.concatenate([srecs[0, :, 32:36], srecs[1, :, 32:36]],
                               axis=1)
        adst = drec[:, :8]

        ex = _edge_ex(asrc[src], adst[dst], ea[:, None] * we[None, :],
                      B.reshape(1, H))
        msg = (h[src].reshape(E, H, C) * ex[:, :, None]).reshape(E, OUT)
        num = jax.ops.segment_sum(msg, dst, num_segments=NP)
        den = jax.ops.segment_sum(ex, dst, num_segments=NP)

        acc = jnp.stack([
            jnp.concatenate([num[:, :32], den[:, :4]], axis=1),
            jnp.concatenate([num[:, 32:], den[:, 4:]], axis=1),
        ])
        xp, eam_new = _finalize(acc, srecs, drec, eam, sea,
                                we.reshape(1, H), B.reshape(1, H),
                                bias.reshape(1, OUT), with_loop)
        if not with_loop:
            eam = eam_new

    return xp[:N]
